# baseline (speedup 1.0000x reference)
import functools

import jax
import jax.numpy as jnp
from jax import lax
from jax.experimental import pallas as pl
from jax.experimental.pallas import tpu as pltpu
from jax.experimental.pallas import tpu_sc as plsc

N, E, D, H, O = 10000, 320000, 128, 64, 64
NP = 10240
NC, NS = 2, 16
BATCH = 128
W = 80
NB0, NB1 = 112, 48
NBMAX = max(NB0, NB1)
EP = NS * (NB0 + NB1) * BATCH
ROWS_PT = NP // NS
NBUF = 4


def _sc_body(y_hbm, src0_hbm, dst0_hbm, src1_hbm, dst1_hbm, zrows_hbm,
             sums_hbm,
             src_v, dst_v, buf0, buf1, buf2, buf3, acc,
             sem0, sem1, sem2, sem3):
  cid = lax.axis_index("c")
  sid = lax.axis_index("s")
  nb = lax.select(cid == 0, jnp.int32(NB0), jnp.int32(NB1))
  bufs = [buf0, buf1, buf2, buf3]
  sems = [sem0, sem1, sem2, sem3]

  pltpu.sync_copy(zrows_hbm, acc.at[pl.ds(sid * ROWS_PT, ROWS_PT)])

  @pl.when(cid == 0)
  def _():
    pltpu.sync_copy(src0_hbm.at[sid], src_v.at[pl.ds(0, NB0)])
    pltpu.sync_copy(dst0_hbm.at[sid], dst_v.at[pl.ds(0, NB0)])

  @pl.when(cid == 1)
  def _():
    pltpu.sync_copy(src1_hbm.at[sid], src_v.at[pl.ds(0, NB1)])
    pltpu.sync_copy(dst1_hbm.at[sid], dst_v.at[pl.ds(0, NB1)])

  plsc.subcore_barrier()

  for k in range(NBUF):
    pltpu.async_copy(y_hbm.at[src_v.at[k]], bufs[k], sems[k])

  def _quad(i, carry):
    for k in range(NBUF):
      b = NBUF * i + k
      pltpu.make_async_copy(y_hbm.at[src_v.at[b]], bufs[k], sems[k]).wait()
      pltpu.sync_copy(bufs[k], acc.at[dst_v.at[b]], add=True)

      @pl.when(b + NBUF < nb)
      def _():
        pltpu.async_copy(y_hbm.at[src_v.at[b + NBUF]], bufs[k], sems[k])
    return carry

  lax.fori_loop(0, nb // NBUF, _quad, 0)
  plsc.subcore_barrier()

  pltpu.sync_copy(acc.at[pl.ds(sid * ROWS_PT, ROWS_PT)],
                  sums_hbm.at[cid, pl.ds(sid * ROWS_PT, ROWS_PT)])


_sc_segment_mean_parts = functools.partial(
    pl.kernel,
    out_type=jax.ShapeDtypeStruct((NC, NP, W), jnp.float32),
    mesh=plsc.VectorSubcoreMesh(core_axis_name="c", subcore_axis_name="s"),
    compiler_params=pltpu.CompilerParams(use_tc_tiling_on_sc=False),
    scratch_types=[
        pltpu.VMEM((NBMAX, BATCH), jnp.int32),
        pltpu.VMEM((NBMAX, BATCH), jnp.int32),
        pltpu.VMEM((BATCH, W), jnp.float32),
        pltpu.VMEM((BATCH, W), jnp.float32),
        pltpu.VMEM((BATCH, W), jnp.float32),
        pltpu.VMEM((BATCH, W), jnp.float32),
        pltpu.VMEM_SHARED((NP, W), jnp.float32),
        pltpu.SemaphoreType.DMA,
        pltpu.SemaphoreType.DMA,
        pltpu.SemaphoreType.DMA,
        pltpu.SemaphoreType.DMA,
    ],
)(_sc_body)


def _mm_body(x_ref, wl_ref, wr_ref, y_ref, z_ref):
  xb = x_ref[...]
  h = jnp.dot(xb, wl_ref[...], preferred_element_type=jnp.float32)
  tail = (lax.broadcasted_iota(jnp.int32, (xb.shape[0], W - H), 1) == 0)
  y_ref[...] = jnp.concatenate([h, tail.astype(jnp.float32)], axis=1)
  z_ref[...] = jnp.dot(xb, wr_ref[...], preferred_element_type=jnp.float32)


def _tc_in_proj(x, W_l, W_r):
  blk = N // 10
  return pl.pallas_call(
      _mm_body,
      grid=(10,),
      in_specs=[
          pl.BlockSpec((blk, D), lambda i: (i, 0)),
          pl.BlockSpec((D, H), lambda i: (0, 0)),
          pl.BlockSpec((D, H), lambda i: (0, 0)),
      ],
      out_specs=[
          pl.BlockSpec((blk, W), lambda i: (i, 0)),
          pl.BlockSpec((blk, H), lambda i: (i, 0)),
      ],
      out_shape=[
          jax.ShapeDtypeStruct((N, W), jnp.float32),
          jax.ShapeDtypeStruct((N, H), jnp.float32),
      ],
      compiler_params=pltpu.CompilerParams(
          dimension_semantics=("parallel",)),
  )(x, W_l, W_r)


def _out_body(sums_ref, z_ref, bl_ref, wlin_ref, blin_ref, o_ref):
  s = sums_ref[0] + sums_ref[1]
  c = s[:, H:H + 1]
  mean = s[:, :H] / jnp.maximum(c, 1.0)
  h = mean + bl_ref[...] + z_ref[...]
  h = jnp.where(h > 0.0, h, jnp.exp(jnp.minimum(h, 0.0)) - 1.0)
  o_ref[...] = (jnp.dot(h, wlin_ref[...], preferred_element_type=jnp.float32)
                + blin_ref[...])


def _tc_out_proj(sums, z, b_l, W_lin, b_lin):
  blk = N // 10
  return pl.pallas_call(
      _out_body,
      grid=(10,),
      in_specs=[
          pl.BlockSpec((NC, blk, W), lambda i: (0, i, 0)),
          pl.BlockSpec((blk, H), lambda i: (i, 0)),
          pl.BlockSpec((1, H), lambda i: (0, 0)),
          pl.BlockSpec((H, O), lambda i: (0, 0)),
          pl.BlockSpec((1, O), lambda i: (0, 0)),
      ],
      out_specs=pl.BlockSpec((blk, O), lambda i: (i, 0)),
      out_shape=jax.ShapeDtypeStruct((N, O), jnp.float32),
      compiler_params=pltpu.CompilerParams(
          dimension_semantics=("parallel",)),
  )(sums, z, b_l.reshape(1, H), W_lin, b_lin.reshape(1, O))


def kernel(x, edge_index, W_l, b_l, W_r, W_lin, b_lin):
  y80, z = _tc_in_proj(x, W_l, W_r)

  pad_e = EP - E
  e0 = NS * NB0 * BATCH
  src_f = jnp.concatenate([edge_index[0], jnp.zeros((pad_e,), jnp.int32)])
  dst_f = jnp.concatenate([edge_index[1], jnp.full((pad_e,), N, jnp.int32)])
  src0 = src_f[:e0].reshape(NS, NB0, BATCH)
  dst0 = dst_f[:e0].reshape(NS, NB0, BATCH)
  src1 = src_f[e0:].reshape(NS, NB1, BATCH)
  dst1 = dst_f[e0:].reshape(NS, NB1, BATCH)

  zrows = jnp.zeros((ROWS_PT, W), jnp.float32)
  sums = _sc_segment_mean_parts(y80, src0, dst0, src1, dst1, zrows)

  return _tc_out_proj(sums, z, b_l, W_lin, b_lin)


# speedup vs baseline: 1.3496x; 1.3496x over previous
import functools

import jax
import jax.numpy as jnp
from jax import lax
from jax.experimental import pallas as pl
from jax.experimental.pallas import tpu as pltpu
from jax.experimental.pallas import tpu_sc as plsc

N, E, D, H, O = 10000, 320000, 128, 64, 64
NP = 10240
NC, NS = 2, 16
BATCH = 128
W = 80
NB0, NB1 = 124, 36
NBMAX = max(NB0, NB1)
EP = NS * (NB0 + NB1) * BATCH
ROWS_PT = NP // NS
NBUF = 4


def _sc_body(y_hbm, src0_hbm, dst0_hbm, src1_hbm, dst1_hbm, zrows_hbm,
             sums_hbm,
             src_v, dst_v, buf0, buf1, buf2, buf3, acc,
             sem0, sem1, sem2, sem3):
  cid = lax.axis_index("c")
  sid = lax.axis_index("s")
  bufs = [buf0, buf1, buf2, buf3]
  sems = [sem0, sem1, sem2, sem3]

  pltpu.sync_copy(zrows_hbm, acc.at[pl.ds(sid * ROWS_PT, ROWS_PT)])

  @pl.when(cid == 0)
  def _():
    pltpu.sync_copy(src0_hbm.at[sid], src_v.at[pl.ds(0, NB0)])
    pltpu.sync_copy(dst0_hbm.at[sid], dst_v.at[pl.ds(0, NB0)])

  @pl.when(cid == 1)
  def _():
    pltpu.sync_copy(src1_hbm.at[sid], src_v.at[pl.ds(0, NB1)])
    pltpu.sync_copy(dst1_hbm.at[sid], dst_v.at[pl.ds(0, NB1)])

  plsc.subcore_barrier()

  def _ring(depth, nbatches):
    for k in range(depth):
      pltpu.async_copy(y_hbm.at[src_v.at[k]], bufs[k], sems[k])

    def _step(i, carry):
      for k in range(depth):
        b = depth * i + k
        pltpu.make_async_copy(y_hbm.at[src_v.at[b]], bufs[k], sems[k]).wait()
        pltpu.sync_copy(bufs[k], acc.at[dst_v.at[b]], add=True)

        @pl.when(b + depth < nbatches)
        def _():
          pltpu.async_copy(y_hbm.at[src_v.at[b + depth]], bufs[k], sems[k])
      return carry

    lax.fori_loop(0, nbatches // depth, _step, 0)

  @pl.when(cid == 0)
  def _():
    _ring(NBUF, NB0)

  @pl.when(cid == 1)
  def _():
    _ring(2, NB1)

  plsc.subcore_barrier()

  pltpu.sync_copy(acc.at[pl.ds(sid * ROWS_PT, ROWS_PT)],
                  sums_hbm.at[cid, pl.ds(sid * ROWS_PT, ROWS_PT)])


_sc_segment_mean_parts = functools.partial(
    pl.kernel,
    out_type=jax.ShapeDtypeStruct((NC, NP, W), jnp.float32),
    mesh=plsc.VectorSubcoreMesh(core_axis_name="c", subcore_axis_name="s"),
    compiler_params=pltpu.CompilerParams(use_tc_tiling_on_sc=False),
    scratch_types=[
        pltpu.VMEM((NBMAX, BATCH), jnp.int32),
        pltpu.VMEM((NBMAX, BATCH), jnp.int32),
        pltpu.VMEM((BATCH, W), jnp.float32),
        pltpu.VMEM((BATCH, W), jnp.float32),
        pltpu.VMEM((BATCH, W), jnp.float32),
        pltpu.VMEM((BATCH, W), jnp.float32),
        pltpu.VMEM_SHARED((NP, W), jnp.float32),
        pltpu.SemaphoreType.DMA,
        pltpu.SemaphoreType.DMA,
        pltpu.SemaphoreType.DMA,
        pltpu.SemaphoreType.DMA,
    ],
)(_sc_body)


def _mm_body(x_ref, wl_ref, wr_ref, y_ref, z_ref):
  xb = x_ref[...]
  h = jnp.dot(xb, wl_ref[...], preferred_element_type=jnp.float32)
  tail = (lax.broadcasted_iota(jnp.int32, (xb.shape[0], W - H), 1) == 0)
  y_ref[...] = jnp.concatenate([h, tail.astype(jnp.float32)], axis=1)
  z_ref[...] = jnp.dot(xb, wr_ref[...], preferred_element_type=jnp.float32)


def _tc_in_proj(x, W_l, W_r):
  blk = N // 10
  return pl.pallas_call(
      _mm_body,
      grid=(10,),
      in_specs=[
          pl.BlockSpec((blk, D), lambda i: (i, 0)),
          pl.BlockSpec((D, H), lambda i: (0, 0)),
          pl.BlockSpec((D, H), lambda i: (0, 0)),
      ],
      out_specs=[
          pl.BlockSpec((blk, W), lambda i: (i, 0)),
          pl.BlockSpec((blk, H), lambda i: (i, 0)),
      ],
      out_shape=[
          jax.ShapeDtypeStruct((N, W), jnp.float32),
          jax.ShapeDtypeStruct((N, H), jnp.float32),
      ],
      compiler_params=pltpu.CompilerParams(
          dimension_semantics=("parallel",)),
  )(x, W_l, W_r)


def _out_body(sums_ref, z_ref, bl_ref, wlin_ref, blin_ref, o_ref):
  s = sums_ref[0] + sums_ref[1]
  c = s[:, H:H + 1]
  mean = s[:, :H] / jnp.maximum(c, 1.0)
  h = mean + bl_ref[...] + z_ref[...]
  h = jnp.where(h > 0.0, h, jnp.exp(jnp.minimum(h, 0.0)) - 1.0)
  o_ref[...] = (jnp.dot(h, wlin_ref[...], preferred_element_type=jnp.float32)
                + blin_ref[...])


def _tc_out_proj(sums, z, b_l, W_lin, b_lin):
  blk = N // 10
  return pl.pallas_call(
      _out_body,
      grid=(10,),
      in_specs=[
          pl.BlockSpec((NC, blk, W), lambda i: (0, i, 0)),
          pl.BlockSpec((blk, H), lambda i: (i, 0)),
          pl.BlockSpec((1, H), lambda i: (0, 0)),
          pl.BlockSpec((H, O), lambda i: (0, 0)),
          pl.BlockSpec((1, O), lambda i: (0, 0)),
      ],
      out_specs=pl.BlockSpec((blk, O), lambda i: (i, 0)),
      out_shape=jax.ShapeDtypeStruct((N, O), jnp.float32),
      compiler_params=pltpu.CompilerParams(
          dimension_semantics=("parallel",)),
  )(sums, z, b_l.reshape(1, H), W_lin, b_lin.reshape(1, O))


def kernel(x, edge_index, W_l, b_l, W_r, W_lin, b_lin):
  y80, z = _tc_in_proj(x, W_l, W_r)

  pad_e = EP - E
  e0 = NS * NB0 * BATCH
  src_f = jnp.concatenate([edge_index[0], jnp.zeros((pad_e,), jnp.int32)])
  dst_f = jnp.concatenate([edge_index[1], jnp.full((pad_e,), N, jnp.int32)])
  src0 = src_f[:e0].reshape(NS, NB0, BATCH)
  dst0 = dst_f[:e0].reshape(NS, NB0, BATCH)
  src1 = src_f[e0:].reshape(NS, NB1, BATCH)
  dst1 = dst_f[e0:].reshape(NS, NB1, BATCH)

  zrows = jnp.zeros((ROWS_PT, W), jnp.float32)
  sums = _sc_segment_mean_parts(y80, src0, dst0, src1, dst1, zrows)

  return _tc_out_proj(sums, z, b_l, W_lin, b_lin)


# speedup vs baseline: 1.4438x; 1.0699x over previous
import functools

import jax
import jax.numpy as jnp
from jax import lax
from jax.experimental import pallas as pl
from jax.experimental.pallas import tpu as pltpu
from jax.experimental.pallas import tpu_sc as plsc

N, E, D, H, O = 10000, 320000, 128, 64, 64
NP = 10240
NC, NS = 2, 16
BATCH = 128
CW = 8
NB0, NB1 = 124, 36
NBMAX = max(NB0, NB1)
EP = NS * (NB0 + NB1) * BATCH
ROWS_PT = NP // NS
NBUF = 4


def _sc_body(y_hbm, src0_hbm, dst0_hbm, src1_hbm, dst1_hbm, zrows_hbm,
             zcnt_hbm, ones_hbm,
             sums_hbm, cnt_hbm,
             src_v, dst_v, buf0, buf1, buf2, buf3, ones_v, acc, cacc,
             sem0, sem1, sem2, sem3, csem):
  cid = lax.axis_index("c")
  sid = lax.axis_index("s")
  bufs = [buf0, buf1, buf2, buf3]
  sems = [sem0, sem1, sem2, sem3]

  pltpu.sync_copy(zrows_hbm, acc.at[pl.ds(sid * ROWS_PT, ROWS_PT)])
  pltpu.sync_copy(zcnt_hbm, cacc.at[pl.ds(sid * ROWS_PT, ROWS_PT)])
  pltpu.sync_copy(ones_hbm, ones_v)

  @pl.when(cid == 0)
  def _():
    pltpu.sync_copy(src0_hbm.at[sid], src_v.at[pl.ds(0, NB0)])
    pltpu.sync_copy(dst0_hbm.at[sid], dst_v.at[pl.ds(0, NB0)])

  @pl.when(cid == 1)
  def _():
    pltpu.sync_copy(src1_hbm.at[sid], src_v.at[pl.ds(0, NB1)])
    pltpu.sync_copy(dst1_hbm.at[sid], dst_v.at[pl.ds(0, NB1)])

  plsc.subcore_barrier()

  def _ring(depth, nbatches, lazy_counts):
    for k in range(depth):
      pltpu.async_copy(y_hbm.at[src_v.at[k]], bufs[k], sems[k])

    def _step(i, carry):
      for k in range(depth):
        b = depth * i + k
        pltpu.make_async_copy(y_hbm.at[src_v.at[b]], bufs[k], sems[k]).wait()
        pltpu.sync_copy(bufs[k], acc.at[dst_v.at[b]], add=True)
        if lazy_counts:
          pltpu.async_copy(ones_v, cacc.at[dst_v.at[b]], csem, add=True)

          @pl.when(i > 0)
          def _():
            pltpu.make_async_copy(ones_v, cacc.at[dst_v.at[b]], csem).wait()
        else:
          pltpu.sync_copy(ones_v, cacc.at[dst_v.at[b]], add=True)

        @pl.when(b + depth < nbatches)
        def _():
          pltpu.async_copy(y_hbm.at[src_v.at[b + depth]], bufs[k], sems[k])
      return carry

    lax.fori_loop(0, nbatches // depth, _step, 0)
    if lazy_counts:
      for k in range(depth):
        pltpu.make_async_copy(ones_v, cacc.at[dst_v.at[0]], csem).wait()

  @pl.when(cid == 0)
  def _():
    _ring(NBUF, NB0, True)

  @pl.when(cid == 1)
  def _():
    _ring(2, NB1, False)

  plsc.subcore_barrier()

  pltpu.sync_copy(acc.at[pl.ds(sid * ROWS_PT, ROWS_PT)],
                  sums_hbm.at[cid, pl.ds(sid * ROWS_PT, ROWS_PT)])
  pltpu.sync_copy(cacc.at[pl.ds(sid * ROWS_PT, ROWS_PT)],
                  cnt_hbm.at[cid, pl.ds(sid * ROWS_PT, ROWS_PT)])


_sc_segment_mean_parts = functools.partial(
    pl.kernel,
    out_type=[
        jax.ShapeDtypeStruct((NC, NP, H), jnp.float32),
        jax.ShapeDtypeStruct((NC, NP, CW), jnp.float32),
    ],
    mesh=plsc.VectorSubcoreMesh(core_axis_name="c", subcore_axis_name="s"),
    compiler_params=pltpu.CompilerParams(use_tc_tiling_on_sc=False),
    scratch_types=[
        pltpu.VMEM((NBMAX, BATCH), jnp.int32),
        pltpu.VMEM((NBMAX, BATCH), jnp.int32),
        pltpu.VMEM((BATCH, H), jnp.float32),
        pltpu.VMEM((BATCH, H), jnp.float32),
        pltpu.VMEM((BATCH, H), jnp.float32),
        pltpu.VMEM((BATCH, H), jnp.float32),
        pltpu.VMEM((BATCH, CW), jnp.float32),
        pltpu.VMEM_SHARED((NP, H), jnp.float32),
        pltpu.VMEM_SHARED((NP, CW), jnp.float32),
        pltpu.SemaphoreType.DMA,
        pltpu.SemaphoreType.DMA,
        pltpu.SemaphoreType.DMA,
        pltpu.SemaphoreType.DMA,
        pltpu.SemaphoreType.DMA,
    ],
)(_sc_body)


def _mm_body(x_ref, wl_ref, wr_ref, y_ref, z_ref):
  xb = x_ref[...]
  y_ref[...] = jnp.dot(xb, wl_ref[...], preferred_element_type=jnp.float32)
  z_ref[...] = jnp.dot(xb, wr_ref[...], preferred_element_type=jnp.float32)


def _tc_in_proj(x, W_l, W_r):
  blk = N // 10
  return pl.pallas_call(
      _mm_body,
      grid=(10,),
      in_specs=[
          pl.BlockSpec((blk, D), lambda i: (i, 0)),
          pl.BlockSpec((D, H), lambda i: (0, 0)),
          pl.BlockSpec((D, H), lambda i: (0, 0)),
      ],
      out_specs=[
          pl.BlockSpec((blk, H), lambda i: (i, 0)),
          pl.BlockSpec((blk, H), lambda i: (i, 0)),
      ],
      out_shape=[
          jax.ShapeDtypeStruct((N, H), jnp.float32),
          jax.ShapeDtypeStruct((N, H), jnp.float32),
      ],
      compiler_params=pltpu.CompilerParams(
          dimension_semantics=("parallel",)),
  )(x, W_l, W_r)


def _out_body(sums_ref, cnt_ref, z_ref, bl_ref, wlin_ref, blin_ref, o_ref):
  s = sums_ref[0] + sums_ref[1]
  c = (cnt_ref[0] + cnt_ref[1])[:, 0:1]
  mean = s / jnp.maximum(c, 1.0)
  h = mean + bl_ref[...] + z_ref[...]
  h = jnp.where(h > 0.0, h, jnp.exp(jnp.minimum(h, 0.0)) - 1.0)
  o_ref[...] = (jnp.dot(h, wlin_ref[...], preferred_element_type=jnp.float32)
                + blin_ref[...])


def _tc_out_proj(sums, cnts, z, b_l, W_lin, b_lin):
  blk = N // 10
  return pl.pallas_call(
      _out_body,
      grid=(10,),
      in_specs=[
          pl.BlockSpec((NC, blk, H), lambda i: (0, i, 0)),
          pl.BlockSpec((NC, blk, CW), lambda i: (0, i, 0)),
          pl.BlockSpec((blk, H), lambda i: (i, 0)),
          pl.BlockSpec((1, H), lambda i: (0, 0)),
          pl.BlockSpec((H, O), lambda i: (0, 0)),
          pl.BlockSpec((1, O), lambda i: (0, 0)),
      ],
      out_specs=pl.BlockSpec((blk, O), lambda i: (i, 0)),
      out_shape=jax.ShapeDtypeStruct((N, O), jnp.float32),
      compiler_params=pltpu.CompilerParams(
          dimension_semantics=("parallel",)),
  )(sums, cnts, z, b_l.reshape(1, H), W_lin, b_lin.reshape(1, O))


def kernel(x, edge_index, W_l, b_l, W_r, W_lin, b_lin):
  y, z = _tc_in_proj(x, W_l, W_r)

  pad_e = EP - E
  e0 = NS * NB0 * BATCH
  src_f = jnp.concatenate([edge_index[0], jnp.zeros((pad_e,), jnp.int32)])
  dst_f = jnp.concatenate([edge_index[1], jnp.full((pad_e,), N, jnp.int32)])
  src0 = src_f[:e0].reshape(NS, NB0, BATCH)
  dst0 = dst_f[:e0].reshape(NS, NB0, BATCH)
  src1 = src_f[e0:].reshape(NS, NB1, BATCH)
  dst1 = dst_f[e0:].reshape(NS, NB1, BATCH)

  zrows = jnp.zeros((ROWS_PT, H), jnp.float32)
  zcnt = jnp.zeros((ROWS_PT, CW), jnp.float32)
  ones_rows = jnp.zeros((BATCH, CW), jnp.float32).at[:, 0].set(1.0)
  sums, cnts = _sc_segment_mean_parts(y, src0, dst0, src1, dst1,
                                      zrows, zcnt, ones_rows)

  return _tc_out_proj(sums, cnts, z, b_l, W_lin, b_lin)
